# baseline (speedup 1.0000x reference)
import functools

import jax
import jax.numpy as jnp
from jax.experimental import pallas as pl
from jax.experimental.pallas import tpu as pltpu

_NUM_CLASSES = 19
_PAIR = 2

_VMEM = 48 * 1024 * 1024


def _cp(*sem):
    return pltpu.CompilerParams(dimension_semantics=sem,
                                vmem_limit_bytes=_VMEM)


def _pool_store(r, hp_ref, o_ref, *, bp, width, cout):
    r = r.reshape(bp, 2, width, cout)
    hp_ref[...] = jnp.maximum(r[:, 0], r[:, 1]).reshape(bp * width, cout)
    wo = width // 2
    ev = hp_ref[pl.ds(0, bp * wo, 2), :]
    od = hp_ref[pl.ds(1, bp * wo, 2), :]
    o_ref[...] = jnp.maximum(ev, od).astype(o_ref.dtype)


def _c1_body(c_ref, w_ref, b_ref, o_ref, hp_ref, *, bp, width, cout):
    acc = jnp.dot(c_ref[...], w_ref[...], preferred_element_type=jnp.float32)
    r = jnp.maximum(acc + b_ref[...], 0.0)
    _pool_store(r, hp_ref, o_ref, bp=bp, width=width, cout=cout)


def _conv1(x_nhwc, w, b, *, bp):
    n, h, wdt, cin = x_nhwc.shape
    cout = w.shape[-1]
    ho, wo = h // 2, wdt // 2
    xp = jnp.pad(x_nhwc, ((0, 0), (1, 1), (1, 1), (0, 0)))
    cols = jnp.concatenate(
        [xp[:, dy:dy + h, dx:dx + wdt, :] for dy in range(3) for dx in range(3)],
        axis=-1).reshape(n * h * wdt, 9 * cin)
    k = 9 * cin
    out = pl.pallas_call(
        functools.partial(_c1_body, bp=bp, width=wdt, cout=cout),
        out_shape=jax.ShapeDtypeStruct((n * ho * wo, cout), jnp.bfloat16),
        grid=(n * ho // bp,),
        in_specs=[pl.BlockSpec((2 * bp * wdt, k), lambda i: (i, 0)),
                  pl.BlockSpec((k, cout), lambda i: (0, 0)),
                  pl.BlockSpec((1, cout), lambda i: (0, 0))],
        out_specs=pl.BlockSpec((bp * wo, cout), lambda i: (i, 0)),
        scratch_shapes=[pltpu.VMEM((bp * wdt, cout), jnp.float32)],
        compiler_params=_cp("parallel"),
    )(cols, w.reshape(k, cout), b.reshape(1, cout))
    return out.reshape(n, ho, wo, cout)


def _dx_body(x_hbm, w_ref, b_ref, o_ref, xbuf, hp_ref, sem,
             *, bp, hpad, width, kin, cout):
    n = pl.program_id(0)
    i = pl.program_id(1)
    nb = pl.num_programs(1)
    rows = 2 * bp + 2
    slot = i & 1

    def fetch(step, s):
        r0 = n * hpad + step * (2 * bp)
        pltpu.make_async_copy(x_hbm.at[pl.ds(r0, rows)],
                              xbuf.at[s], sem.at[s]).start()

    @pl.when(i == 0)
    def _():
        fetch(i, slot)

    pltpu.make_async_copy(x_hbm.at[pl.ds(0, rows)],
                          xbuf.at[slot], sem.at[slot]).wait()

    @pl.when(i + 1 < nb)
    def _():
        fetch(i + 1, 1 - slot)

    m = 2 * bp * width
    xv = xbuf[slot]
    acc = jnp.dot(xv[0:2 * bp].reshape(m, kin), w_ref[0],
                  preferred_element_type=jnp.float32)
    acc += jnp.dot(xv[1:2 * bp + 1].reshape(m, kin), w_ref[1],
                   preferred_element_type=jnp.float32)
    acc += jnp.dot(xv[2:2 * bp + 2].reshape(m, kin), w_ref[2],
                   preferred_element_type=jnp.float32)
    r = jnp.maximum(acc + b_ref[...], 0.0)
    _pool_store(r, hp_ref, o_ref.at[0], bp=bp, width=width, cout=cout)


def _conv_dx(x_nhwc, w, b, *, bp):
    n, h, wdt, cin = x_nhwc.shape
    cout = w.shape[-1]
    ho, wo = h // 2, wdt // 2
    hpad = h + 2
    kin = 3 * cin
    xp = jnp.pad(x_nhwc, ((0, 0), (1, 1), (1, 1), (0, 0)))
    slab = jnp.concatenate([xp[:, :, dx:dx + wdt, :] for dx in range(3)],
                           axis=-1).reshape(n * hpad, wdt, kin)
    wd = w.reshape(3, kin, cout)
    rows = 2 * bp + 2
    out = pl.pallas_call(
        functools.partial(_dx_body, bp=bp, hpad=hpad,
                          width=wdt, kin=kin, cout=cout),
        out_shape=jax.ShapeDtypeStruct((n, ho * wo, cout), jnp.bfloat16),
        grid=(n, ho // bp),
        in_specs=[pl.BlockSpec(memory_space=pl.ANY),
                  pl.BlockSpec((3, kin, cout), lambda nn, i: (0, 0, 0)),
                  pl.BlockSpec((1, cout), lambda nn, i: (0, 0))],
        out_specs=pl.BlockSpec((1, bp * wo, cout), lambda nn, i: (nn, i, 0)),
        scratch_shapes=[pltpu.VMEM((2, rows, wdt, kin), jnp.bfloat16),
                        pltpu.VMEM((bp * wdt, cout), jnp.float32),
                        pltpu.SemaphoreType.DMA((2,))],
        compiler_params=_cp("parallel", "arbitrary"),
    )(slab, wd, b.reshape(1, cout))
    return out.reshape(n, ho, wo, cout)


def _halo_body(x_hbm, w_ref, b_ref, o_ref, xbuf, hp_ref, sem,
               *, bp, hpad, width, cin, cout):
    n = pl.program_id(0)
    i = pl.program_id(1)
    nb = pl.num_programs(1)
    rows = 2 * bp + 2
    slot = i & 1

    def fetch(step, s):
        r0 = n * hpad + step * (2 * bp)
        pltpu.make_async_copy(x_hbm.at[pl.ds(r0, rows)],
                              xbuf.at[s], sem.at[s]).start()

    @pl.when(i == 0)
    def _():
        fetch(i, slot)

    pltpu.make_async_copy(x_hbm.at[pl.ds(0, rows)],
                          xbuf.at[slot], sem.at[slot]).wait()

    @pl.when(i + 1 < nb)
    def _():
        fetch(i + 1, 1 - slot)

    xs = xbuf[slot]
    acc = jnp.zeros((2 * bp * width, cout), jnp.float32)
    for dy in range(3):
        for dx in range(3):
            tap = xs[dy:dy + 2 * bp, dx:dx + width, :].reshape(2 * bp * width, cin)
            acc += jnp.dot(tap, w_ref[dy * 3 + dx],
                           preferred_element_type=jnp.float32)
    r = jnp.maximum(acc + b_ref[...], 0.0)
    _pool_store(r, hp_ref, o_ref.at[0], bp=bp, width=width, cout=cout)


def _conv_halo(x_nhwc, w, b, *, bp):
    n, h, wdt, cin = x_nhwc.shape
    cout = w.shape[-1]
    ho, wo = h // 2, wdt // 2
    hpad, wpad = h + 2, wdt + 2
    xp = jnp.pad(x_nhwc, ((0, 0), (1, 1), (1, 1), (0, 0))).reshape(n * hpad, wpad, cin)
    rows = 2 * bp + 2
    out = pl.pallas_call(
        functools.partial(_halo_body, bp=bp, hpad=hpad,
                          width=wdt, cin=cin, cout=cout),
        out_shape=jax.ShapeDtypeStruct((n, ho * wo, cout), jnp.bfloat16),
        grid=(n, ho // bp),
        in_specs=[pl.BlockSpec(memory_space=pl.ANY),
                  pl.BlockSpec((9, cin, cout), lambda nn, i: (0, 0, 0)),
                  pl.BlockSpec((1, cout), lambda nn, i: (0, 0))],
        out_specs=pl.BlockSpec((1, bp * wo, cout), lambda nn, i: (nn, i, 0)),
        scratch_shapes=[pltpu.VMEM((2, rows, wpad, cin), jnp.bfloat16),
                        pltpu.VMEM((bp * wdt, cout), jnp.float32),
                        pltpu.SemaphoreType.DMA((2,))],
        compiler_params=_cp("parallel", "arbitrary"),
    )(xp, w.reshape(9, cin, cout), b.reshape(1, cout))
    return out.reshape(n, ho, wo, cout)


def _fc_body(x_ref, w_ref, b_ref, o_ref, *, relu):
    k = pl.program_id(1)

    @pl.when(k == 0)
    def _():
        o_ref[...] = jnp.zeros_like(o_ref)

    o_ref[...] += jnp.dot(x_ref[...], w_ref[...],
                          preferred_element_type=jnp.float32)

    @pl.when(k == pl.num_programs(1) - 1)
    def _():
        r = o_ref[...] + b_ref[...]
        if relu:
            r = jnp.maximum(r, 0.0)
        o_ref[...] = r


def _fc(x, w, b, *, relu, tk, tn=None):
    m, kdim = x.shape
    nout = w.shape[1]
    if tn is None:
        tn = nout
    return pl.pallas_call(
        functools.partial(_fc_body, relu=relu),
        out_shape=jax.ShapeDtypeStruct((m, nout), jnp.float32),
        grid=(nout // tn, kdim // tk),
        in_specs=[pl.BlockSpec((m, tk), lambda j, k: (0, k)),
                  pl.BlockSpec((tk, tn), lambda j, k: (k, j)),
                  pl.BlockSpec((1, tn), lambda j, k: (0, j))],
        out_specs=pl.BlockSpec((m, tn), lambda j, k: (0, j)),
        compiler_params=_cp("parallel", "arbitrary"),
    )(x, w, b.reshape(1, nout))


def kernel(x, c0w, c0b, c1w, c1b, c2w, c2b, c3w, c3b, c4w, c4b,
           fc1_w, fc1_b, fc2_w, fc2_b):
    xh = jnp.transpose(x, (0, 2, 3, 1)).astype(jnp.bfloat16)
    a = _conv1(xh, c0w, c0b, bp=40)
    a = _conv_dx(a, c1w, c1b, bp=40)
    a = _conv_dx(a, c2w, c2b, bp=16)
    a = _conv_halo(a, c3w, c3b, bp=20)
    a = _conv_halo(a, c4w, c4b, bp=10)
    n = a.shape[0]
    feat = jnp.transpose(a, (0, 3, 1, 2)).reshape(n // _PAIR, -1)
    h = _fc(feat, fc1_w, fc1_b, relu=True, tk=6400, tn=256)
    return _fc(h.astype(jnp.bfloat16), fc2_w, fc2_b, relu=False, tk=512)


# speedup vs baseline: 1.0368x; 1.0368x over previous
import functools

import jax
import jax.numpy as jnp
from jax.experimental import pallas as pl
from jax.experimental.pallas import tpu as pltpu

_NUM_CLASSES = 19
_PAIR = 2

_VMEM = 48 * 1024 * 1024


def _cp(*sem):
    return pltpu.CompilerParams(dimension_semantics=sem,
                                vmem_limit_bytes=_VMEM)


def _pool_store(r, hp_ref, o_ref, *, bp, width, cout):
    r = r.reshape(bp, 2, width, cout)
    hp_ref[...] = jnp.maximum(r[:, 0], r[:, 1]).reshape(bp * width, cout)
    wo = width // 2
    ev = hp_ref[pl.ds(0, bp * wo, 2), :]
    od = hp_ref[pl.ds(1, bp * wo, 2), :]
    o_ref[...] = jnp.maximum(ev, od).astype(o_ref.dtype)


def _c1_body(c_ref, w_ref, b_ref, o_ref, hp_ref, *, bp, width, cout):
    acc = jnp.dot(c_ref[...], w_ref[...], preferred_element_type=jnp.float32)
    r = jnp.maximum(acc + b_ref[...], 0.0)
    _pool_store(r, hp_ref, o_ref, bp=bp, width=width, cout=cout)


def _conv1(x_nhwc, w, b, *, bp):
    n, h, wdt, cin = x_nhwc.shape
    cout = w.shape[-1]
    ho, wo = h // 2, wdt // 2
    xp = jnp.pad(x_nhwc, ((0, 0), (1, 1), (1, 1), (0, 0)))
    cols = jnp.concatenate(
        [xp[:, dy:dy + h, dx:dx + wdt, :] for dy in range(3) for dx in range(3)],
        axis=-1).reshape(n * h * wdt, 9 * cin)
    k = 9 * cin
    out = pl.pallas_call(
        functools.partial(_c1_body, bp=bp, width=wdt, cout=cout),
        out_shape=jax.ShapeDtypeStruct((n * ho * wo, cout), jnp.bfloat16),
        grid=(n * ho // bp,),
        in_specs=[pl.BlockSpec((2 * bp * wdt, k), lambda i: (i, 0)),
                  pl.BlockSpec((k, cout), lambda i: (0, 0)),
                  pl.BlockSpec((1, cout), lambda i: (0, 0))],
        out_specs=pl.BlockSpec((bp * wo, cout), lambda i: (i, 0)),
        scratch_shapes=[pltpu.VMEM((bp * wdt, cout), jnp.float32)],
        compiler_params=_cp("parallel"),
    )(cols, w.reshape(k, cout), b.reshape(1, cout))
    return out.reshape(n, ho, wo, cout)


def _dx_body(x_hbm, w_ref, b_ref, o_ref, xbuf, hp_ref, sem,
             *, bp, hpad, width, kin, cout):
    n = pl.program_id(0)
    i = pl.program_id(1)
    nb = pl.num_programs(1)
    rows = 2 * bp + 2
    slot = i & 1

    def fetch(step, s):
        r0 = n * hpad + step * (2 * bp)
        pltpu.make_async_copy(x_hbm.at[pl.ds(r0, rows)],
                              xbuf.at[s], sem.at[s]).start()

    @pl.when(i == 0)
    def _():
        fetch(i, slot)

    pltpu.make_async_copy(x_hbm.at[pl.ds(0, rows)],
                          xbuf.at[slot], sem.at[slot]).wait()

    @pl.when(i + 1 < nb)
    def _():
        fetch(i + 1, 1 - slot)

    m = 2 * bp * width
    xv = xbuf[slot]
    acc = jnp.dot(xv[0:2 * bp].reshape(m, kin), w_ref[0],
                  preferred_element_type=jnp.float32)
    acc += jnp.dot(xv[1:2 * bp + 1].reshape(m, kin), w_ref[1],
                   preferred_element_type=jnp.float32)
    acc += jnp.dot(xv[2:2 * bp + 2].reshape(m, kin), w_ref[2],
                   preferred_element_type=jnp.float32)
    r = jnp.maximum(acc + b_ref[...], 0.0)
    _pool_store(r, hp_ref, o_ref.at[0], bp=bp, width=width, cout=cout)


def _conv_dx(x_nhwc, w, b, *, bp):
    n, h, wdt, cin = x_nhwc.shape
    cout = w.shape[-1]
    ho, wo = h // 2, wdt // 2
    hpad = h + 2
    kin = 3 * cin
    xp = jnp.pad(x_nhwc, ((0, 0), (1, 1), (1, 1), (0, 0)))
    slab = jnp.concatenate([xp[:, :, dx:dx + wdt, :] for dx in range(3)],
                           axis=-1).reshape(n * hpad, wdt, kin)
    wd = w.reshape(3, kin, cout)
    rows = 2 * bp + 2
    out = pl.pallas_call(
        functools.partial(_dx_body, bp=bp, hpad=hpad,
                          width=wdt, kin=kin, cout=cout),
        out_shape=jax.ShapeDtypeStruct((n, ho * wo, cout), jnp.bfloat16),
        grid=(n, ho // bp),
        in_specs=[pl.BlockSpec(memory_space=pl.ANY),
                  pl.BlockSpec((3, kin, cout), lambda nn, i: (0, 0, 0)),
                  pl.BlockSpec((1, cout), lambda nn, i: (0, 0))],
        out_specs=pl.BlockSpec((1, bp * wo, cout), lambda nn, i: (nn, i, 0)),
        scratch_shapes=[pltpu.VMEM((2, rows, wdt, kin), jnp.bfloat16),
                        pltpu.VMEM((bp * wdt, cout), jnp.float32),
                        pltpu.SemaphoreType.DMA((2,))],
        compiler_params=_cp("parallel", "arbitrary"),
    )(slab, wd, b.reshape(1, cout))
    return out.reshape(n, ho, wo, cout)


def _halo_body(x_hbm, w_ref, b_ref, o_ref, xbuf, hp_ref, sem,
               *, bp, hpad, width, cin, cout):
    n = pl.program_id(0)
    i = pl.program_id(1)
    nb = pl.num_programs(1)
    rows = 2 * bp + 2
    slot = i & 1

    def fetch(step, s):
        r0 = n * hpad + step * (2 * bp)
        pltpu.make_async_copy(x_hbm.at[pl.ds(r0, rows)],
                              xbuf.at[s], sem.at[s]).start()

    @pl.when(i == 0)
    def _():
        fetch(i, slot)

    pltpu.make_async_copy(x_hbm.at[pl.ds(0, rows)],
                          xbuf.at[slot], sem.at[slot]).wait()

    @pl.when(i + 1 < nb)
    def _():
        fetch(i + 1, 1 - slot)

    xs = xbuf[slot]
    acc = jnp.zeros((2 * bp * width, cout), jnp.float32)
    for dy in range(3):
        for dx in range(3):
            tap = xs[dy:dy + 2 * bp, dx:dx + width, :].reshape(2 * bp * width, cin)
            acc += jnp.dot(tap, w_ref[dy * 3 + dx],
                           preferred_element_type=jnp.float32)
    r = jnp.maximum(acc + b_ref[...], 0.0)
    _pool_store(r, hp_ref, o_ref.at[0], bp=bp, width=width, cout=cout)


def _conv_halo(x_nhwc, w, b, *, bp):
    n, h, wdt, cin = x_nhwc.shape
    cout = w.shape[-1]
    ho, wo = h // 2, wdt // 2
    hpad, wpad = h + 2, wdt + 2
    xp = jnp.pad(x_nhwc, ((0, 0), (1, 1), (1, 1), (0, 0))).reshape(n * hpad, wpad, cin)
    rows = 2 * bp + 2
    out = pl.pallas_call(
        functools.partial(_halo_body, bp=bp, hpad=hpad,
                          width=wdt, cin=cin, cout=cout),
        out_shape=jax.ShapeDtypeStruct((n, ho * wo, cout), jnp.bfloat16),
        grid=(n, ho // bp),
        in_specs=[pl.BlockSpec(memory_space=pl.ANY),
                  pl.BlockSpec((9, cin, cout), lambda nn, i: (0, 0, 0)),
                  pl.BlockSpec((1, cout), lambda nn, i: (0, 0))],
        out_specs=pl.BlockSpec((1, bp * wo, cout), lambda nn, i: (nn, i, 0)),
        scratch_shapes=[pltpu.VMEM((2, rows, wpad, cin), jnp.bfloat16),
                        pltpu.VMEM((bp * wdt, cout), jnp.float32),
                        pltpu.SemaphoreType.DMA((2,))],
        compiler_params=_cp("parallel", "arbitrary"),
    )(xp, w.reshape(9, cin, cout), b.reshape(1, cout))
    return out.reshape(n, ho, wo, cout)


def _fc_body(x_ref, w_ref, b_ref, o_ref, *, relu):
    k = pl.program_id(1)

    @pl.when(k == 0)
    def _():
        o_ref[...] = jnp.zeros_like(o_ref)

    o_ref[...] += jnp.dot(x_ref[...], w_ref[...],
                          preferred_element_type=jnp.float32)

    @pl.when(k == pl.num_programs(1) - 1)
    def _():
        r = o_ref[...] + b_ref[...]
        if relu:
            r = jnp.maximum(r, 0.0)
        o_ref[...] = r


def _fc(x, w, b, *, relu, tk, tn=None):
    m, kdim = x.shape
    nout = w.shape[1]
    if tn is None:
        tn = nout
    return pl.pallas_call(
        functools.partial(_fc_body, relu=relu),
        out_shape=jax.ShapeDtypeStruct((m, nout), jnp.float32),
        grid=(nout // tn, kdim // tk),
        in_specs=[pl.BlockSpec((m, tk), lambda j, k: (0, k)),
                  pl.BlockSpec((tk, tn), lambda j, k: (k, j)),
                  pl.BlockSpec((1, tn), lambda j, k: (0, j))],
        out_specs=pl.BlockSpec((m, tn), lambda j, k: (0, j)),
        compiler_params=_cp("parallel", "arbitrary"),
    )(x, w, b.reshape(1, nout))


def kernel(x, c0w, c0b, c1w, c1b, c2w, c2b, c3w, c3b, c4w, c4b,
           fc1_w, fc1_b, fc2_w, fc2_b):
    xh = jnp.transpose(x, (0, 2, 3, 1)).astype(jnp.bfloat16)
    a = _conv1(xh, c0w, c0b, bp=40)
    a = _conv_dx(a, c1w, c1b, bp=40)
    a = _conv_halo(a, c2w, c2b, bp=16)
    a = _conv_halo(a, c3w, c3b, bp=20)
    a = _conv_halo(a, c4w, c4b, bp=10)
    n = a.shape[0]
    feat = jnp.transpose(a, (0, 3, 1, 2)).reshape(n // _PAIR, -1)
    h = _fc(feat, fc1_w, fc1_b, relu=True, tk=6400, tn=256)
    return _fc(h.astype(jnp.bfloat16), fc2_w, fc2_b, relu=False, tk=512)
